# baseline (speedup 1.0000x reference)
import functools

import jax
import jax.numpy as jnp
from jax import lax
from jax.experimental import pallas as pl
from jax.experimental.pallas import tpu as pltpu
from jax.experimental.pallas import tpu_sc as plsc

_BB = 32


def _stats_body(q_ref, eps_ref, idx_ref, lse_ref):
    q = q_ref[...]
    m = jnp.max(q, axis=-1, keepdims=True)
    idx = jnp.argmax(q, axis=-1).astype(jnp.int32)
    s = jnp.sum(jnp.exp(q - m), axis=-1, keepdims=True)
    lse = m + jnp.log(s)
    eps_ref[...] = jnp.broadcast_to(m, (_BB, 128))
    idx_ref[...] = jnp.broadcast_to(idx[:, None], (_BB, 128))
    lse_ref[...] = jnp.broadcast_to(lse, (_BB, 128))


def _stats_call(q_mean):
    b, n = q_mean.shape
    return pl.pallas_call(
        _stats_body,
        grid=(b // _BB,),
        in_specs=[pl.BlockSpec((_BB, n), lambda i: (i, 0))],
        out_specs=[
            pl.BlockSpec((_BB, 128), lambda i: (i, 0)),
            pl.BlockSpec((_BB, 128), lambda i: (i, 0)),
            pl.BlockSpec((_BB, 128), lambda i: (i, 0)),
        ],
        out_shape=[
            jax.ShapeDtypeStruct((b, 128), jnp.float32),
            jax.ShapeDtypeStruct((b, 128), jnp.int32),
            jax.ShapeDtypeStruct((b, 128), jnp.float32),
        ],
    )(q_mean)


def _lp_body(q_ref, lse_ref, lp_ref):
    lp_ref[...] = q_ref[...] - lse_ref[:, 0:1]


def _lp_call(q_mean, lse_bcast):
    b, n = q_mean.shape
    return pl.pallas_call(
        _lp_body,
        grid=(b // _BB,),
        in_specs=[
            pl.BlockSpec((_BB, n), lambda i: (i, 0)),
            pl.BlockSpec((_BB, 128), lambda i: (i, 0)),
        ],
        out_specs=pl.BlockSpec((_BB, n), lambda i: (i, 0)),
        out_shape=jax.ShapeDtypeStruct((b, n), jnp.float32),
    )(q_mean, lse_bcast)


def _sc_gather_call(table_t, eps_bcast, ind_bcast):
    b = table_t.shape[0]
    a = table_t.shape[1]
    n_workers = 16
    rpw = b // n_workers
    mesh = plsc.VectorSubcoreMesh(
        core_axis_name="c", subcore_axis_name="s", num_cores=1)

    @functools.partial(
        pl.kernel,
        out_type=[
            jax.ShapeDtypeStruct((b, a), jnp.float32),
            jax.ShapeDtypeStruct((b,), jnp.float32),
        ],
        mesh=mesh,
        compiler_params=pltpu.CompilerParams(
            needs_layout_passes=False,
            disable_bounds_checks=True,
            disable_semaphore_checks=True,
            skip_device_barrier=True,
        ),
        scratch_types=[
            pltpu.VMEM((rpw, 128), jnp.int32),
            pltpu.VMEM((rpw, 128), jnp.float32),
            pltpu.VMEM((16,), jnp.float32),
            pltpu.VMEM((rpw, a, 128), jnp.float32),
            pltpu.VMEM((rpw, a), jnp.float32),
            pltpu.SemaphoreType.DMA,
        ],
    )
    def gather_kernel(table_hbm, eps_hbm, ind_hbm, out_hbm, out_eps_hbm,
                      iidx_t, eps_t, eps_v, rows_v, final_v, sem):
        wid = lax.axis_index("s") + lax.axis_index("c")

        @pl.when(wid < n_workers)
        def _():
            base = wid * rpw
            pltpu.sync_copy(ind_hbm.at[pl.ds(base, rpw)], iidx_t)
            pltpu.sync_copy(eps_hbm.at[pl.ds(base, rpw)], eps_t)
            row16 = lax.iota(jnp.int32, 16) & (rpw - 1)
            zero16 = jnp.zeros((16,), jnp.int32)
            iv = plsc.load_gather(iidx_t, [row16, zero16])
            cv = iv >> 7
            lv = iv & 127
            eps_v[...] = plsc.load_gather(eps_t, [row16, zero16])
            pltpu.sync_copy(eps_v.at[pl.ds(0, rpw)],
                            out_eps_hbm.at[pl.ds(base, rpw)])
            copies = []
            for r in range(rpw):
                cp = pltpu.make_async_copy(
                    table_hbm.at[base + r, :, pl.ds(cv[r] * 128, 128)],
                    rows_v.at[r], sem)
                cp.start()
                copies.append(cp)
            for cp in copies:
                cp.wait()
            lane_iota = lax.iota(jnp.int32, 16)
            for r in range(rpw):
                j = jnp.full((16,), lv[r], jnp.int32)
                sel = plsc.load_gather(
                    rows_v, [jnp.full((16,), r, jnp.int32), lane_iota, j])
                final_v[r, :] = sel
            pltpu.sync_copy(final_v, out_hbm.at[pl.ds(base, rpw)])

    return gather_kernel(table_t, eps_bcast, ind_bcast)


def kernel(q_mean, q_stddev, actions):
    del q_stddev
    eps, ind, lse = _stats_call(q_mean)
    actions_t = jnp.transpose(actions, (0, 2, 1))
    best_action, best_eps = _sc_gather_call(actions_t, eps, ind)
    log_probs = _lp_call(q_mean, lse)
    return (log_probs, best_action, best_eps)


# speedup vs baseline: 1.0369x; 1.0369x over previous
import functools

import jax
import jax.numpy as jnp
from jax import lax
from jax.experimental import pallas as pl
from jax.experimental.pallas import tpu as pltpu
from jax.experimental.pallas import tpu_sc as plsc

_BB = 32


def _stats_body(q_ref, eps_ref, idx_ref):
    q = q_ref[...]
    m = jnp.max(q, axis=-1, keepdims=True)
    idx = jnp.argmax(q, axis=-1).astype(jnp.int32)
    eps_ref[...] = jnp.broadcast_to(m, (_BB, 128))
    idx_ref[...] = jnp.broadcast_to(idx[:, None], (_BB, 128))


def _stats_call(q_mean):
    b, n = q_mean.shape
    return pl.pallas_call(
        _stats_body,
        grid=(b // _BB,),
        in_specs=[pl.BlockSpec((_BB, n), lambda i: (i, 0))],
        out_specs=[
            pl.BlockSpec((_BB, 128), lambda i: (i, 0)),
            pl.BlockSpec((_BB, 128), lambda i: (i, 0)),
        ],
        out_shape=[
            jax.ShapeDtypeStruct((b, 128), jnp.float32),
            jax.ShapeDtypeStruct((b, 128), jnp.int32),
        ],
    )(q_mean)


def _lp_body(q_ref, eps_ref, lp_ref):
    q = q_ref[...]
    m = eps_ref[:, 0:1]
    s = jnp.sum(jnp.exp(q - m), axis=-1, keepdims=True)
    lp_ref[...] = q - (m + jnp.log(s))


def _lp_call(q_mean, eps_bcast):
    b, n = q_mean.shape
    return pl.pallas_call(
        _lp_body,
        grid=(b // _BB,),
        in_specs=[
            pl.BlockSpec((_BB, n), lambda i: (i, 0)),
            pl.BlockSpec((_BB, 128), lambda i: (i, 0)),
        ],
        out_specs=pl.BlockSpec((_BB, n), lambda i: (i, 0)),
        out_shape=jax.ShapeDtypeStruct((b, n), jnp.float32),
    )(q_mean, eps_bcast)


def _sc_gather_call(table_t, eps_bcast, ind_bcast):
    b = table_t.shape[0]
    a = table_t.shape[1]
    n_workers = 16
    rpw = b // n_workers
    mesh = plsc.VectorSubcoreMesh(
        core_axis_name="c", subcore_axis_name="s", num_cores=1)

    @functools.partial(
        pl.kernel,
        out_type=[
            jax.ShapeDtypeStruct((b, a), jnp.float32),
            jax.ShapeDtypeStruct((b,), jnp.float32),
        ],
        mesh=mesh,
        compiler_params=pltpu.CompilerParams(
            needs_layout_passes=False,
            disable_bounds_checks=True,
            disable_semaphore_checks=True,
            skip_device_barrier=True,
        ),
        scratch_types=[
            pltpu.VMEM((rpw, 128), jnp.int32),
            pltpu.VMEM((rpw, 128), jnp.float32),
            pltpu.VMEM((16,), jnp.float32),
            pltpu.VMEM((rpw, a, 128), jnp.float32),
            pltpu.VMEM((rpw, a), jnp.float32),
            pltpu.SemaphoreType.DMA,
        ],
    )
    def gather_kernel(table_hbm, eps_hbm, ind_hbm, out_hbm, out_eps_hbm,
                      iidx_t, eps_t, eps_v, rows_v, final_v, sem):
        wid = lax.axis_index("s") + lax.axis_index("c")

        @pl.when(wid < n_workers)
        def _():
            base = wid * rpw
            pltpu.sync_copy(ind_hbm.at[pl.ds(base, rpw)], iidx_t)
            pltpu.sync_copy(eps_hbm.at[pl.ds(base, rpw)], eps_t)
            row16 = lax.iota(jnp.int32, 16) & (rpw - 1)
            zero16 = jnp.zeros((16,), jnp.int32)
            iv = plsc.load_gather(iidx_t, [row16, zero16])
            cv = iv >> 7
            lv = iv & 127
            eps_v[...] = plsc.load_gather(eps_t, [row16, zero16])
            pltpu.sync_copy(eps_v.at[pl.ds(0, rpw)],
                            out_eps_hbm.at[pl.ds(base, rpw)])
            copies = []
            for r in range(rpw):
                cp = pltpu.make_async_copy(
                    table_hbm.at[base + r, :, pl.ds(cv[r] * 128, 128)],
                    rows_v.at[r], sem)
                cp.start()
                copies.append(cp)
            for cp in copies:
                cp.wait()
            lane_iota = lax.iota(jnp.int32, 16)
            for r in range(rpw):
                j = jnp.full((16,), lv[r], jnp.int32)
                sel = plsc.load_gather(
                    rows_v, [jnp.full((16,), r, jnp.int32), lane_iota, j])
                final_v[r, :] = sel
            pltpu.sync_copy(final_v, out_hbm.at[pl.ds(base, rpw)])

    return gather_kernel(table_t, eps_bcast, ind_bcast)


def kernel(q_mean, q_stddev, actions):
    del q_stddev
    eps, ind = _stats_call(q_mean)
    actions_t = jnp.transpose(actions, (0, 2, 1))
    best_action, best_eps = _sc_gather_call(actions_t, eps, ind)
    log_probs = _lp_call(q_mean, eps)
    return (log_probs, best_action, best_eps)


# speedup vs baseline: 1.1172x; 1.0774x over previous
import functools

import jax
import jax.numpy as jnp
from jax import lax
from jax.experimental import pallas as pl
from jax.experimental.pallas import tpu as pltpu
from jax.experimental.pallas import tpu_sc as plsc

_BB = 32


def _dense_body(q_ref, lp_ref, eps_ref, idx_ref):
    q = q_ref[...]
    m = jnp.max(q, axis=-1, keepdims=True)
    idx = jnp.argmax(q, axis=-1).astype(jnp.int32)
    s = jnp.sum(jnp.exp(q - m), axis=-1, keepdims=True)
    lse = m + jnp.log(s)
    lp_ref[...] = q - lse
    eps_ref[...] = jnp.broadcast_to(m, (_BB, 128))
    idx_ref[...] = jnp.broadcast_to(idx[:, None], (_BB, 128))


def _dense_call(q_mean):
    b, n = q_mean.shape
    return pl.pallas_call(
        _dense_body,
        grid=(b // _BB,),
        in_specs=[pl.BlockSpec((_BB, n), lambda i: (i, 0))],
        out_specs=[
            pl.BlockSpec((_BB, n), lambda i: (i, 0)),
            pl.BlockSpec((_BB, 128), lambda i: (i, 0)),
            pl.BlockSpec((_BB, 128), lambda i: (i, 0)),
        ],
        out_shape=[
            jax.ShapeDtypeStruct((b, n), jnp.float32),
            jax.ShapeDtypeStruct((b, 128), jnp.float32),
            jax.ShapeDtypeStruct((b, 128), jnp.int32),
        ],
    )(q_mean)


def _sc_gather_call(table_t, eps_bcast, ind_bcast):
    b = table_t.shape[0]
    a = table_t.shape[1]
    n_workers = 16
    rpw = b // n_workers
    mesh = plsc.VectorSubcoreMesh(
        core_axis_name="c", subcore_axis_name="s", num_cores=1)

    @functools.partial(
        pl.kernel,
        out_type=[
            jax.ShapeDtypeStruct((b, a), jnp.float32),
            jax.ShapeDtypeStruct((b,), jnp.float32),
        ],
        mesh=mesh,
        compiler_params=pltpu.CompilerParams(
            needs_layout_passes=False,
            disable_bounds_checks=True,
            disable_semaphore_checks=True,
            skip_device_barrier=True,
        ),
        scratch_types=[
            pltpu.VMEM((rpw, 128), jnp.int32),
            pltpu.VMEM((rpw, 128), jnp.float32),
            pltpu.VMEM((16,), jnp.float32),
            pltpu.VMEM((rpw, a, 128), jnp.float32),
            pltpu.VMEM((rpw, a), jnp.float32),
            pltpu.SemaphoreType.DMA,
        ],
    )
    def gather_kernel(table_hbm, eps_hbm, ind_hbm, out_hbm, out_eps_hbm,
                      iidx_t, eps_t, eps_v, rows_v, final_v, sem):
        wid = lax.axis_index("s") + lax.axis_index("c")

        @pl.when(wid < n_workers)
        def _():
            base = wid * rpw
            pltpu.sync_copy(ind_hbm.at[pl.ds(base, rpw)], iidx_t)
            pltpu.sync_copy(eps_hbm.at[pl.ds(base, rpw)], eps_t)
            row16 = lax.iota(jnp.int32, 16) & (rpw - 1)
            zero16 = jnp.zeros((16,), jnp.int32)
            iv = plsc.load_gather(iidx_t, [row16, zero16])
            cv = iv >> 7
            lv = iv & 127
            eps_v[...] = plsc.load_gather(eps_t, [row16, zero16])
            pltpu.sync_copy(eps_v.at[pl.ds(0, rpw)],
                            out_eps_hbm.at[pl.ds(base, rpw)])
            copies = []
            for r in range(rpw):
                cp = pltpu.make_async_copy(
                    table_hbm.at[base + r, :, pl.ds(cv[r] * 128, 128)],
                    rows_v.at[r], sem)
                cp.start()
                copies.append(cp)
            for cp in copies:
                cp.wait()
            lane_iota = lax.iota(jnp.int32, 16)
            for r in range(rpw):
                j = jnp.full((16,), lv[r], jnp.int32)
                sel = plsc.load_gather(
                    rows_v, [jnp.full((16,), r, jnp.int32), lane_iota, j])
                final_v[r, :] = sel
            pltpu.sync_copy(final_v, out_hbm.at[pl.ds(base, rpw)])

    return gather_kernel(table_t, eps_bcast, ind_bcast)


def kernel(q_mean, q_stddev, actions):
    del q_stddev
    log_probs, eps, ind = _dense_call(q_mean)
    actions_t = jnp.transpose(actions, (0, 2, 1))
    best_action, best_eps = _sc_gather_call(actions_t, eps, ind)
    return (log_probs, best_action, best_eps)


# speedup vs baseline: 1.1454x; 1.0252x over previous
import functools

import jax
import jax.numpy as jnp
from jax import lax
from jax.experimental import pallas as pl
from jax.experimental.pallas import tpu as pltpu
from jax.experimental.pallas import tpu_sc as plsc

_BB = 32


def _dense_body(q_ref, lp_ref, eps_ref, idx_ref):
    q = q_ref[...]
    n = q.shape[-1]
    m = jnp.max(q, axis=-1, keepdims=True)
    ii = lax.broadcasted_iota(jnp.int32, q.shape, 1)
    idx = jnp.min(jnp.where(q == m, ii, jnp.int32(n)), axis=-1)
    s = jnp.sum(jnp.exp(q - m), axis=-1, keepdims=True)
    lse = m + jnp.log(s)
    lp_ref[...] = q - lse
    eps_ref[...] = jnp.broadcast_to(m, (_BB, 128))
    idx_ref[...] = jnp.broadcast_to(idx[:, None], (_BB, 128))


def _dense_call(q_mean):
    b, n = q_mean.shape
    return pl.pallas_call(
        _dense_body,
        grid=(b // _BB,),
        in_specs=[pl.BlockSpec((_BB, n), lambda i: (i, 0))],
        out_specs=[
            pl.BlockSpec((_BB, n), lambda i: (i, 0)),
            pl.BlockSpec((_BB, 128), lambda i: (i, 0)),
            pl.BlockSpec((_BB, 128), lambda i: (i, 0)),
        ],
        out_shape=[
            jax.ShapeDtypeStruct((b, n), jnp.float32),
            jax.ShapeDtypeStruct((b, 128), jnp.float32),
            jax.ShapeDtypeStruct((b, 128), jnp.int32),
        ],
    )(q_mean)


def _sc_gather_call(table_t, eps_bcast, ind_bcast):
    b = table_t.shape[0]
    a = table_t.shape[1]
    n_workers = 16
    rpw = b // n_workers
    mesh = plsc.VectorSubcoreMesh(
        core_axis_name="c", subcore_axis_name="s", num_cores=1)

    @functools.partial(
        pl.kernel,
        out_type=[
            jax.ShapeDtypeStruct((b, a), jnp.float32),
            jax.ShapeDtypeStruct((b,), jnp.float32),
        ],
        mesh=mesh,
        compiler_params=pltpu.CompilerParams(
            needs_layout_passes=False,
            disable_bounds_checks=True,
            disable_semaphore_checks=True,
            skip_device_barrier=True,
        ),
        scratch_types=[
            pltpu.VMEM((rpw, 128), jnp.int32),
            pltpu.VMEM((rpw, 128), jnp.float32),
            pltpu.VMEM((16,), jnp.float32),
            pltpu.VMEM((rpw, a, 128), jnp.float32),
            pltpu.VMEM((rpw, a), jnp.float32),
            pltpu.SemaphoreType.DMA,
        ],
    )
    def gather_kernel(table_hbm, eps_hbm, ind_hbm, out_hbm, out_eps_hbm,
                      iidx_t, eps_t, eps_v, rows_v, final_v, sem):
        wid = lax.axis_index("s") + lax.axis_index("c")

        @pl.when(wid < n_workers)
        def _():
            base = wid * rpw
            pltpu.sync_copy(ind_hbm.at[pl.ds(base, rpw)], iidx_t)
            pltpu.sync_copy(eps_hbm.at[pl.ds(base, rpw)], eps_t)
            row16 = lax.iota(jnp.int32, 16) & (rpw - 1)
            zero16 = jnp.zeros((16,), jnp.int32)
            iv = plsc.load_gather(iidx_t, [row16, zero16])
            cv = iv >> 7
            lv = iv & 127
            eps_v[...] = plsc.load_gather(eps_t, [row16, zero16])
            pltpu.sync_copy(eps_v.at[pl.ds(0, rpw)],
                            out_eps_hbm.at[pl.ds(base, rpw)])
            copies = []
            for r in range(rpw):
                cp = pltpu.make_async_copy(
                    table_hbm.at[base + r, :, pl.ds(cv[r] * 128, 128)],
                    rows_v.at[r], sem)
                cp.start()
                copies.append(cp)
            for cp in copies:
                cp.wait()
            lane_iota = lax.iota(jnp.int32, 16)
            for r in range(rpw):
                j = jnp.full((16,), lv[r], jnp.int32)
                sel = plsc.load_gather(
                    rows_v, [jnp.full((16,), r, jnp.int32), lane_iota, j])
                final_v[r, :] = sel
            pltpu.sync_copy(final_v, out_hbm.at[pl.ds(base, rpw)])

    return gather_kernel(table_t, eps_bcast, ind_bcast)


def kernel(q_mean, q_stddev, actions):
    del q_stddev
    log_probs, eps, ind = _dense_call(q_mean)
    actions_t = jnp.transpose(actions, (0, 2, 1))
    best_action, best_eps = _sc_gather_call(actions_t, eps, ind)
    return (log_probs, best_action, best_eps)
